# baseline (speedup 1.0000x reference)
import functools
import jax
import jax.numpy as jnp
from jax import lax
from jax.experimental import pallas as pl


def _router_block(x_ref, w_ref, b_ref, out_ref, mask_ref, *, num_experts):
    lt = lax.dot_general(w_ref[...], x_ref[...], (((1,), (1,)), ((), ())),
                         preferred_element_type=jnp.float32)
    lt = lt + b_ref[...]
    idx = jax.lax.broadcasted_iota(jnp.int32, lt.shape, 0)
    m1 = jnp.max(lt, axis=0, keepdims=True)
    i1 = jnp.min(jnp.where(lt == m1, idx, num_experts), axis=0, keepdims=True)
    l2 = jnp.where(idx == i1, -jnp.inf, lt)
    m2 = jnp.max(l2, axis=0, keepdims=True)
    i2 = jnp.min(jnp.where(l2 == m2, idx, num_experts), axis=0, keepdims=True)
    z = jnp.sum(jnp.exp(lt - m1), axis=0, keepdims=True)
    a = 1.0 / (1.0 + jnp.exp((jnp.exp(m2 - m1) - 1.0) / z))
    sel1 = idx == i1
    sel2 = idx == i2
    r_t = jnp.where(sel1, a, jnp.where(sel2, 1.0 - a, 0.0))
    mask_t = jnp.logical_or(sel1, sel2).astype(jnp.float32)
    out_ref[...] = r_t.T
    mask_ref[...] = mask_t.T


def kernel(x, W, b):
    B, T, C = x.shape
    E = W.shape[0]
    tokens = B * T
    blk = 1024
    xf = x.reshape(tokens, C)
    out, mask = pl.pallas_call(
        functools.partial(_router_block, num_experts=E),
        grid=(tokens // blk,),
        in_specs=[
            pl.BlockSpec((blk, C), lambda i: (i, 0)),
            pl.BlockSpec((E, C), lambda i: (0, 0)),
            pl.BlockSpec((E, 1), lambda i: (0, 0)),
        ],
        out_specs=[
            pl.BlockSpec((blk, E), lambda i: (i, 0)),
            pl.BlockSpec((blk, E), lambda i: (i, 0)),
        ],
        out_shape=[
            jax.ShapeDtypeStruct((tokens, E), jnp.float32),
            jax.ShapeDtypeStruct((tokens, E), jnp.float32),
        ],
    )(xf, W, b.reshape(E, 1))
    return out.reshape(B, T, E), mask.reshape(B, T, E)


# speedup vs baseline: 1.3015x; 1.3015x over previous
import functools

import jax
import jax.numpy as jnp
from jax import lax
from jax.experimental import pallas as pl


def _router_block(x_ref, w_ref, b_ref, out_ref, mask_ref, *, num_experts):
    lt = lax.dot_general(
        w_ref[...], x_ref[...], (((1,), (1,)), ((), ())),
        preferred_element_type=jnp.float32)
    lt = lt + b_ref[...]
    idx = lax.broadcasted_iota(jnp.int32, lt.shape, 0)
    m1 = jnp.max(lt, axis=0, keepdims=True)
    i1 = jnp.min(jnp.where(lt == m1, idx, num_experts), axis=0, keepdims=True)
    l2 = jnp.where(idx == i1, -jnp.inf, lt)
    m2 = jnp.max(l2, axis=0, keepdims=True)
    i2 = jnp.min(jnp.where(l2 == m2, idx, num_experts), axis=0, keepdims=True)
    z = jnp.sum(jnp.exp(lt - m1), axis=0, keepdims=True)
    a = 1.0 / (1.0 + jnp.exp((jnp.exp(m2 - m1) - 1.0) / z))
    sel1 = idx == i1
    sel2 = idx == i2
    out_ref[...] = jnp.where(sel1, a, jnp.where(sel2, 1.0 - a, 0.0))
    mask_ref[...] = jnp.logical_or(sel1, sel2).astype(jnp.float32)


def kernel(x, W, b):
    B, T, C = x.shape
    E = W.shape[0]
    tokens = B * T
    blk = 1024
    xf = x.reshape(tokens, C)
    out_t, mask_t = pl.pallas_call(
        functools.partial(_router_block, num_experts=E),
        grid=(tokens // blk,),
        in_specs=[
            pl.BlockSpec((blk, C), lambda i: (i, 0)),
            pl.BlockSpec((E, C), lambda i: (0, 0)),
            pl.BlockSpec((E, 1), lambda i: (0, 0)),
        ],
        out_specs=[
            pl.BlockSpec((E, blk), lambda i: (0, i)),
            pl.BlockSpec((E, blk), lambda i: (0, i)),
        ],
        out_shape=[
            jax.ShapeDtypeStruct((E, tokens), jnp.float32),
            jax.ShapeDtypeStruct((E, tokens), jnp.float32),
        ],
    )(xf, W, b.reshape(E, 1))
    return (out_t.T.reshape(B, T, E), mask_t.T.reshape(B, T, E))
